# baseline (speedup 1.0000x reference)
import functools

import jax
import jax.numpy as jnp
from jax import lax
from jax.experimental import pallas as pl
from jax.experimental.pallas import tpu as pltpu
from jax.experimental.pallas import tpu_sc as plsc

N_CAND = 8
B, S, D = 2, 2048, 1024
SLAB_ROWS = B * S
TOTAL_ROWS = N_CAND * SLAB_ROWS

NW = 32
ROWS_PER_W = SLAB_ROWS // NW
CHUNK_ROWS = 16
N_CHUNKS = ROWS_PER_W // CHUNK_ROWS
LANES = 16
COL_GROUPS = D // LANES


def _avg_pair_impl(in_ref, dsel_ref, out_ref, dvec, a0, b0, a1, b1,
                   sa0, sb0, sa1, sb1, so0, so1):
    cid = lax.axis_index("c")
    sid = lax.axis_index("s")
    wid = sid * 2 + cid

    pltpu.sync_copy(dsel_ref, dvec)
    d = dvec[...][0]
    arow = d * (2 * SLAB_ROWS) + wid * ROWS_PER_W
    brow = arow + SLAB_ROWS
    orow = wid * ROWS_PER_W

    bufs = ((a0, b0, sa0, sb0, so0), (a1, b1, sa1, sb1, so1))

    def start_in(g):
        a, b, sa, sb, _ = bufs[g % 2]
        off = g * CHUNK_ROWS
        da = pltpu.async_copy(in_ref.at[pl.ds(arow + off, CHUNK_ROWS)], a, sa)
        db = pltpu.async_copy(in_ref.at[pl.ds(brow + off, CHUNK_ROWS)], b, sb)
        return da, db

    half = jnp.float32(0.5)
    out_dmas = [None, None]
    pend = start_in(0)
    for g in range(N_CHUNKS):
        a, b, _, _, so = bufs[g % 2]
        nxt = None
        if g + 1 < N_CHUNKS:
            if out_dmas[(g + 1) % 2] is not None:
                out_dmas[(g + 1) % 2].wait()
                out_dmas[(g + 1) % 2] = None
            nxt = start_in(g + 1)
        pend[0].wait()
        pend[1].wait()

        @plsc.parallel_loop(0, CHUNK_ROWS * COL_GROUPS, step=1, unroll=16)
        def _(i):
            r = i >> 6
            c = (i & (COL_GROUPS - 1)) * LANES
            a[r, pl.ds(c, LANES)] = (
                a[r, pl.ds(c, LANES)] + b[r, pl.ds(c, LANES)]) * half

        out_dmas[g % 2] = pltpu.async_copy(
            a, out_ref.at[pl.ds(orow + g * CHUNK_ROWS, CHUNK_ROWS)], so)
        pend = nxt

    for od in out_dmas:
        if od is not None:
            od.wait()


@functools.lru_cache(maxsize=1)
def _build_avg_pair():
    mesh = plsc.VectorSubcoreMesh(core_axis_name="c", subcore_axis_name="s")
    return pl.kernel(
        _avg_pair_impl,
        out_type=jax.ShapeDtypeStruct((SLAB_ROWS, D), jnp.float32),
        mesh=mesh,
        compiler_params=pltpu.CompilerParams(use_tc_tiling_on_sc=True),
        scratch_types=[
            pltpu.VMEM((LANES,), jnp.int32),
            pltpu.VMEM((CHUNK_ROWS, D), jnp.float32),
            pltpu.VMEM((CHUNK_ROWS, D), jnp.float32),
            pltpu.VMEM((CHUNK_ROWS, D), jnp.float32),
            pltpu.VMEM((CHUNK_ROWS, D), jnp.float32),
            pltpu.SemaphoreType.DMA,
            pltpu.SemaphoreType.DMA,
            pltpu.SemaphoreType.DMA,
            pltpu.SemaphoreType.DMA,
            pltpu.SemaphoreType.DMA,
            pltpu.SemaphoreType.DMA,
        ],
    )


def kernel(inputs, domain_idx):
    rows = inputs.reshape(TOTAL_ROWS, D)
    dsel = jnp.full((LANES,), jnp.asarray(domain_idx, jnp.int32), jnp.int32)
    out = _build_avg_pair()(rows, dsel)
    return out.reshape(B, S, D)


# speedup vs baseline: 1.0129x; 1.0129x over previous
import functools

import jax
import jax.numpy as jnp
from jax import lax
from jax.experimental import pallas as pl
from jax.experimental.pallas import tpu as pltpu
from jax.experimental.pallas import tpu_sc as plsc

N_CAND = 8
B, S, D = 2, 2048, 1024
SLAB_ROWS = B * S
TOTAL_ROWS = N_CAND * SLAB_ROWS

NW = 32
ROWS_PER_W = SLAB_ROWS // NW
CHUNK_ROWS = 16
N_CHUNKS = ROWS_PER_W // CHUNK_ROWS
LANES = 16
COL_GROUPS = D // LANES


def _avg_pair_impl(in_ref, dsel_ref, out_ref, dvec, a0, b0, a1, b1,
                   sa0, sb0, sa1, sb1, so0, so1):
    cid = lax.axis_index("c")
    sid = lax.axis_index("s")
    wid = sid * 2 + cid

    pltpu.sync_copy(dsel_ref, dvec)
    d = dvec[...][0]
    arow = d * (2 * SLAB_ROWS) + wid * ROWS_PER_W
    brow = arow + SLAB_ROWS
    orow = wid * ROWS_PER_W

    bufs = ((a0, b0, sa0, sb0, so0), (a1, b1, sa1, sb1, so1))

    def start_in(g):
        a, b, sa, sb, _ = bufs[g % 2]
        off = g * CHUNK_ROWS
        da = pltpu.async_copy(in_ref.at[pl.ds(arow + off, CHUNK_ROWS)], a, sa)
        db = pltpu.async_copy(in_ref.at[pl.ds(brow + off, CHUNK_ROWS)], b, sb)
        return da, db

    half = jnp.float32(0.5)
    out_dmas = [None, None]
    pend = start_in(0)
    for g in range(N_CHUNKS):
        a, b, _, _, so = bufs[g % 2]
        nxt = None
        if g + 1 < N_CHUNKS:
            if out_dmas[(g + 1) % 2] is not None:
                out_dmas[(g + 1) % 2].wait()
                out_dmas[(g + 1) % 2] = None
            nxt = start_in(g + 1)
        pend[0].wait()
        pend[1].wait()

        @plsc.parallel_loop(0, CHUNK_ROWS * COL_GROUPS, step=1, unroll=4)
        def _(i):
            r = i >> 6
            c = (i & (COL_GROUPS - 1)) * LANES
            a[r, pl.ds(c, LANES)] = (
                a[r, pl.ds(c, LANES)] + b[r, pl.ds(c, LANES)]) * half

        out_dmas[g % 2] = pltpu.async_copy(
            a, out_ref.at[pl.ds(orow + g * CHUNK_ROWS, CHUNK_ROWS)], so)
        pend = nxt

    for od in out_dmas:
        if od is not None:
            od.wait()


@functools.lru_cache(maxsize=1)
def _build_avg_pair():
    mesh = plsc.VectorSubcoreMesh(core_axis_name="c", subcore_axis_name="s")
    return pl.kernel(
        _avg_pair_impl,
        out_type=jax.ShapeDtypeStruct((SLAB_ROWS, D), jnp.float32),
        mesh=mesh,
        compiler_params=pltpu.CompilerParams(use_tc_tiling_on_sc=True),
        scratch_types=[
            pltpu.VMEM((LANES,), jnp.int32),
            pltpu.VMEM((CHUNK_ROWS, D), jnp.float32),
            pltpu.VMEM((CHUNK_ROWS, D), jnp.float32),
            pltpu.VMEM((CHUNK_ROWS, D), jnp.float32),
            pltpu.VMEM((CHUNK_ROWS, D), jnp.float32),
            pltpu.SemaphoreType.DMA,
            pltpu.SemaphoreType.DMA,
            pltpu.SemaphoreType.DMA,
            pltpu.SemaphoreType.DMA,
            pltpu.SemaphoreType.DMA,
            pltpu.SemaphoreType.DMA,
        ],
    )


def kernel(inputs, domain_idx):
    rows = inputs.reshape(TOTAL_ROWS, D)
    dsel = jnp.full((LANES,), jnp.asarray(domain_idx, jnp.int32), jnp.int32)
    out = _build_avg_pair()(rows, dsel)
    return out.reshape(B, S, D)


# speedup vs baseline: 1.0227x; 1.0096x over previous
import functools

import jax
import jax.numpy as jnp
from jax import lax
from jax.experimental import pallas as pl
from jax.experimental.pallas import tpu as pltpu
from jax.experimental.pallas import tpu_sc as plsc

N_CAND = 8
B, S, D = 2, 2048, 1024
SLAB_ROWS = B * S
TOTAL_ROWS = N_CAND * SLAB_ROWS

NW = 32
ROWS_PER_W = SLAB_ROWS // NW
CHUNK_ROWS = 16
N_CHUNKS = ROWS_PER_W // CHUNK_ROWS
NSET = 3
LANES = 16
COL_GROUPS = D // LANES


def _avg_pair_impl(in_ref, dsel_ref, out_ref, dvec,
                   a0, b0, a1, b1, a2, b2,
                   sa0, sb0, sa1, sb1, sa2, sb2, so0, so1, so2):
    cid = lax.axis_index("c")
    sid = lax.axis_index("s")
    wid = sid * 2 + cid

    pltpu.sync_copy(dsel_ref, dvec)
    d = dvec[...][0]
    arow = d * (2 * SLAB_ROWS) + wid * ROWS_PER_W
    brow = arow + SLAB_ROWS
    orow = wid * ROWS_PER_W

    sets = ((a0, b0, sa0, sb0, so0),
            (a1, b1, sa1, sb1, so1),
            (a2, b2, sa2, sb2, so2))

    def start_in(g):
        a, b, sa, sb, _ = sets[g % NSET]
        off = g * CHUNK_ROWS
        da = pltpu.async_copy(in_ref.at[pl.ds(arow + off, CHUNK_ROWS)], a, sa)
        db = pltpu.async_copy(in_ref.at[pl.ds(brow + off, CHUNK_ROWS)], b, sb)
        return da, db

    half = jnp.float32(0.5)
    pend = [None] * (N_CHUNKS + 1)
    out_dmas = [None] * N_CHUNKS
    pend[0] = start_in(0)
    pend[1] = start_in(1)
    for g in range(N_CHUNKS):
        a, b, _, _, so = sets[g % NSET]
        if g + 2 < N_CHUNKS:
            if g >= 1 and out_dmas[g - 1] is not None:
                out_dmas[g - 1].wait()
                out_dmas[g - 1] = None
            pend[g + 2] = start_in(g + 2)
        pend[g][0].wait()
        pend[g][1].wait()

        @plsc.parallel_loop(0, CHUNK_ROWS * COL_GROUPS, step=1, unroll=4)
        def _(i):
            r = i >> 6
            c = (i & (COL_GROUPS - 1)) * LANES
            a[r, pl.ds(c, LANES)] = (
                a[r, pl.ds(c, LANES)] + b[r, pl.ds(c, LANES)]) * half

        out_dmas[g] = pltpu.async_copy(
            a, out_ref.at[pl.ds(orow + g * CHUNK_ROWS, CHUNK_ROWS)], so)

    for od in out_dmas:
        if od is not None:
            od.wait()


@functools.lru_cache(maxsize=1)
def _build_avg_pair():
    mesh = plsc.VectorSubcoreMesh(core_axis_name="c", subcore_axis_name="s")
    return pl.kernel(
        _avg_pair_impl,
        out_type=jax.ShapeDtypeStruct((SLAB_ROWS, D), jnp.float32),
        mesh=mesh,
        compiler_params=pltpu.CompilerParams(use_tc_tiling_on_sc=True),
        scratch_types=(
            [pltpu.VMEM((LANES,), jnp.int32)]
            + [pltpu.VMEM((CHUNK_ROWS, D), jnp.float32)] * (2 * NSET)
            + [pltpu.SemaphoreType.DMA] * (3 * NSET)
        ),
    )


def kernel(inputs, domain_idx):
    rows = inputs.reshape(TOTAL_ROWS, D)
    dsel = jnp.full((LANES,), jnp.asarray(domain_idx, jnp.int32), jnp.int32)
    out = _build_avg_pair()(rows, dsel)
    return out.reshape(B, S, D)


# speedup vs baseline: 1.0293x; 1.0065x over previous
import functools

import jax
import jax.numpy as jnp
from jax import lax
from jax.experimental import pallas as pl
from jax.experimental.pallas import tpu as pltpu
from jax.experimental.pallas import tpu_sc as plsc

N_CAND = 8
B, S, D = 2, 2048, 1024
SLAB_ROWS = B * S
TOTAL_ROWS = N_CAND * SLAB_ROWS

NW = 32
ROWS_PER_W = SLAB_ROWS // NW
CHUNK_ROWS = 16
N_CHUNKS = ROWS_PER_W // CHUNK_ROWS
NSET = 3
LANES = 16
COL_GROUPS = D // LANES


def _avg_pair_impl(in_ref, dsel_ref, out_ref, dvec,
                   a0, b0, a1, b1, a2, b2,
                   sa0, sb0, sa1, sb1, sa2, sb2, so0, so1, so2):
    cid = lax.axis_index("c")
    sid = lax.axis_index("s")
    wid = sid * 2 + cid

    pltpu.sync_copy(dsel_ref, dvec)
    d = dvec[...][0]
    arow = d * (2 * SLAB_ROWS) + wid * ROWS_PER_W
    brow = arow + SLAB_ROWS
    orow = wid * ROWS_PER_W

    sets = ((a0, b0, sa0, sb0, so0),
            (a1, b1, sa1, sb1, so1),
            (a2, b2, sa2, sb2, so2))

    def start_in(g):
        a, b, sa, sb, _ = sets[g % NSET]
        off = g * CHUNK_ROWS
        da = pltpu.async_copy(in_ref.at[pl.ds(arow + off, CHUNK_ROWS)], a, sa)
        db = pltpu.async_copy(in_ref.at[pl.ds(brow + off, CHUNK_ROWS)], b, sb)
        return da, db

    half = jnp.float32(0.5)
    pend = [None] * (N_CHUNKS + 1)
    out_dmas = [None] * N_CHUNKS
    pend[0] = start_in(0)
    pend[1] = start_in(1)
    for g in range(N_CHUNKS):
        a, b, _, _, so = sets[g % NSET]
        if g + 2 < N_CHUNKS:
            if g >= 1 and out_dmas[g - 1] is not None:
                out_dmas[g - 1].wait()
                out_dmas[g - 1] = None
            pend[g + 2] = start_in(g + 2)
        pend[g][0].wait()
        pend[g][1].wait()

        @plsc.parallel_loop(0, CHUNK_ROWS * COL_GROUPS, step=1, unroll=4)
        def _(i):
            r = i >> 6
            c = (i & (COL_GROUPS - 1)) * LANES
            a[r, pl.ds(c, LANES)] = (
                a[r, pl.ds(c, LANES)] + b[r, pl.ds(c, LANES)]) * half

        out_dmas[g] = pltpu.async_copy(
            a, out_ref.at[pl.ds(orow + g * CHUNK_ROWS, CHUNK_ROWS)], so)

    for od in out_dmas:
        if od is not None:
            od.wait()


@functools.lru_cache(maxsize=1)
def _build_avg_pair():
    mesh = plsc.VectorSubcoreMesh(core_axis_name="c", subcore_axis_name="s")
    return pl.kernel(
        _avg_pair_impl,
        out_type=jax.ShapeDtypeStruct((SLAB_ROWS, D), jnp.float32),
        mesh=mesh,
        compiler_params=pltpu.CompilerParams(
            use_tc_tiling_on_sc=True,
            skip_device_barrier=True,
            disable_bounds_checks=True,
            disable_semaphore_checks=True,
        ),
        scratch_types=(
            [pltpu.VMEM((LANES,), jnp.int32)]
            + [pltpu.VMEM((CHUNK_ROWS, D), jnp.float32)] * (2 * NSET)
            + [pltpu.SemaphoreType.DMA] * (3 * NSET)
        ),
    )


def kernel(inputs, domain_idx):
    rows = inputs.reshape(TOTAL_ROWS, D)
    dsel = jnp.full((LANES,), jnp.asarray(domain_idx, jnp.int32), jnp.int32)
    out = _build_avg_pair()(rows, dsel)
    return out.reshape(B, S, D)
